# baseline (speedup 1.0000x reference)
import functools

import numpy as np
import jax
import jax.numpy as jnp
from jax import lax
from jax.experimental import pallas as pl
from jax.experimental.pallas import tpu as pltpu
from jax.experimental.pallas import tpu_sc as plsc

_MIN_LEN_SEG = 19
_MAX_LEN_SEG = 32
_MAX_LEN_SEQ = 2048
_MAX_LEN_PAD = 4096
_MAX_NUM_SEG = _MAX_LEN_SEQ // _MIN_LEN_SEG + 1
_B = 16
_D = 512

_NC, _NS, _L = 2, 16, 16
_KT = 32
_ZB = 128


_ROT = ((13, 15, 26, 6), (17, 29, 16, 24))


def _fry2(k0, k1, x0, x1):
    x0 = x0.astype(np.uint32).copy()
    x1 = x1.astype(np.uint32).copy()
    ks = [np.uint32(k0), np.uint32(k1),
          np.uint32(np.uint32(k0) ^ np.uint32(k1) ^ np.uint32(0x1BD11BDA))]
    with np.errstate(over="ignore"):
        x0 += ks[0]
        x1 += ks[1]
        for i in range(5):
            for r in _ROT[i % 2]:
                x0 += x1
                x1 = (x1 << np.uint32(r)) | (x1 >> np.uint32(32 - r))
                x1 ^= x0
            x0 += ks[(i + 1) % 3]
            x1 += ks[(i + 2) % 3] + np.uint32(i + 1)
    return x0, x1


def _fry_split(k0, k1, num=2):
    y0, y1 = _fry2(k0, k1, np.zeros(num, np.uint32),
                   np.arange(num, dtype=np.uint32))
    return np.stack([y0, y1], axis=1)


def _fry_bits(k0, k1, n):
    y0, y1 = _fry2(k0, k1, np.zeros(n, np.uint32), np.arange(n, dtype=np.uint32))
    return y0 ^ y1


def _fry_uniform(k0, k1, n):
    bits = _fry_bits(k0, k1, n)
    f = ((bits >> np.uint32(9)) | np.uint32(0x3F800000)).view(np.float32)
    return np.maximum(np.float32(0.0), f - np.float32(1.0))


def _fry_randint(k0, k1, n, minval, maxval):
    keys = _fry_split(k0, k1)
    u = _fry_bits(keys[0, 0], keys[0, 1], n)
    v = _fry_bits(keys[1, 0], keys[1, 1], n)
    span = np.uint32(maxval - minval)
    with np.errstate(over="ignore"):
        mult = ((np.uint32(65536) % span) ** 2) % span
        out = ((u % span) * mult + (v % span)) % span
    return minval + out.astype(np.int32)


@functools.lru_cache(maxsize=None)
def _consts():
    n = _B * _MAX_NUM_SEG
    keys = _fry_split(0, 42)
    scales = _fry_uniform(keys[0, 0], keys[0, 1], n) + np.float32(0.5)
    len_seg = _fry_randint(keys[1, 0], keys[1, 1], n,
                           _MIN_LEN_SEG, _MAX_LEN_SEG).reshape(n, 1)
    j = np.arange(_MAX_LEN_SEG * 2, dtype=np.int32)[None, :]
    idx_scaled = j.astype(np.float32) / scales[:, None]
    idx_fl = np.floor(idx_scaled)
    lam = (idx_scaled - idx_fl).astype(np.float32)
    static_mask = idx_fl < (len_seg.astype(np.float32) - np.float32(1.0))
    off = np.cumsum(len_seg.reshape(_B, -1), axis=-1)
    off = np.pad(off[:, :-1], ((0, 0), (1, 0))).reshape(-1, 1)
    src = (idx_fl + off.astype(np.float32)).astype(np.int32)
    grid = _MAX_NUM_SEG * _MAX_LEN_SEG * 2
    src = src.reshape(_B, grid)
    lam = lam.reshape(_B, grid)
    static_mask = static_mask.reshape(_B, grid)
    rows, lams, counts = [], [], []
    for b in range(_B):
        sel = static_mask[b] & (src[b] < _MAX_LEN_SEQ - 2)
        rows.append(src[b][sel])
        lams.append(lam[b][sel])
        counts.append(int(sel.sum()))
    K = -(-max(counts) // 128) * 128
    sg = np.full((_B, K), 4000, np.int32)
    lm = np.zeros((_B, K), np.float32)
    for b in range(_B):
        sg[b, : counts[b]] = rows[b]
        lm[b, : counts[b]] = lams[b]
    sg = sg + (np.arange(_B, dtype=np.int32) * _MAX_LEN_PAD)[:, None]
    sgi = np.empty((_B, 2 * K), np.int32)
    sgi[:, 0::2] = sg
    sgi[:, 1::2] = sg + 1
    return sg, sgi, lm, K


def kernel(x, len_seq):
    sg_np, sgi_np, lm_np, K = _consts()
    tiles_per_half = (K // _KT) // _NC
    zrows = _MAX_LEN_PAD - K
    zper_half = zrows // _NC

    xf = x.reshape(_B * _MAX_LEN_PAD, _D)
    mesh = plsc.VectorSubcoreMesh(core_axis_name="c", subcore_axis_name="s")

    @functools.partial(
        pl.kernel,
        mesh=mesh,
        compiler_params=pltpu.CompilerParams(needs_layout_passes=False),
        out_type=jax.ShapeDtypeStruct((_B * _MAX_LEN_PAD, _D), jnp.float32),
        scratch_types=[
            pltpu.VMEM((K,), jnp.int32),
            pltpu.VMEM((2 * K,), jnp.int32),
            pltpu.VMEM((K,), jnp.float32),
            pltpu.VMEM((2 * _KT, _D), jnp.float32),
            pltpu.VMEM((2 * _KT, _D), jnp.float32),
            pltpu.VMEM((_KT, _D), jnp.float32),
            pltpu.VMEM((_KT, _D), jnp.float32),
            pltpu.VMEM((_KT, _D), jnp.float32),
            pltpu.VMEM((_L,), jnp.int32),
            pltpu.SemaphoreType.DMA,
            pltpu.SemaphoreType.DMA,
            pltpu.SemaphoreType.DMA,
            pltpu.SemaphoreType.DMA,
            pltpu.SemaphoreType.DMA,
            pltpu.SemaphoreType.DMA,
        ],
    )
    def _k(xf_hbm, lseq_hbm, sfl_hbm, sgi_hbm, lam_hbm, out_hbm,
           sfl_r, sgi_r, lam_r, cb0, cb1, ob0, ob1, zero_v, lseq_v,
           ga0, ga1, wb0, wb1, zsem, csem):
        b = lax.axis_index("s")
        half = lax.axis_index("c")
        pltpu.async_copy(lseq_hbm, lseq_v, csem)
        pltpu.async_copy(sfl_hbm.at[b], sfl_r, csem)
        pltpu.async_copy(sgi_hbm.at[b], sgi_r, csem)
        pltpu.async_copy(lam_hbm.at[b], lam_r, csem)

        zeros16 = jnp.zeros((_L,), jnp.float32)

        def _memset_row(r, c):
            for cidx in range(_D // _L):
                zero_v[r, pl.ds(cidx * _L, _L)] = zeros16
            return c

        lax.fori_loop(0, _KT, _memset_row, 0)

        pltpu.make_async_copy(lseq_hbm, lseq_v, csem).wait()
        pltpu.make_async_copy(sfl_hbm.at[b], sfl_r, csem).wait()
        pltpu.make_async_copy(sgi_hbm.at[b], sgi_r, csem).wait()
        pltpu.make_async_copy(lam_hbm.at[b], lam_r, csem).wait()

        thr = (plsc.load_gather(lseq_v, [jnp.full((_L,), b, jnp.int32)])
               + b * _MAX_LEN_PAD - 1)

        def _cnt(i, acc):
            s16 = sfl_r[pl.ds(i * _L, _L)]
            return acc + jnp.where(s16 < thr, jnp.int32(1), jnp.int32(0))

        nvec = lax.fori_loop(0, K // _L, _cnt, jnp.zeros((_L,), jnp.int32))
        n_b = jnp.sum(nvec)
        stride = _NC * _KT
        nt = jnp.minimum((jnp.maximum(n_b - half * _KT, 0) + (stride - 1))
                         // stride, tiles_per_half)

        obase = b * _MAX_LEN_PAD

        def _ztile(i, c):
            koff = i * stride + half * _KT
            pltpu.async_copy(zero_v, out_hbm.at[pl.ds(obase + koff, _KT)], zsem)
            return c

        lax.fori_loop(nt, tiles_per_half, _ztile, 0)

        zbase = obase + K + half * zper_half
        for j in range(zper_half // _KT):
            pltpu.async_copy(zero_v, out_hbm.at[pl.ds(zbase + j * _KT, _KT)], zsem)

        bufs = ((cb0, ob0, ga0, wb0), (cb1, ob1, ga1, wb1))

        def _issue(i, cbb, ga):
            koff = i * stride + half * _KT
            pltpu.async_copy(xf_hbm.at[sgi_r.at[pl.ds(2 * koff, 2 * _KT)]], cbb, ga)

        def _wait_gather(cbb, ga):
            pltpu.make_async_copy(xf_hbm.at[sgi_r.at[pl.ds(0, 2 * _KT)]], cbb, ga).wait()

        def _wait_wb(obb, wb):
            pltpu.make_async_copy(obb, out_hbm.at[pl.ds(0, _KT)], wb).wait()

        def _blend(i, cbb, obb):
            koff = i * stride + half * _KT

            def _row(k, cc):
                kk = jnp.full((_L,), koff + k, jnp.int32)
                sk = plsc.load_gather(sfl_r, [kk])
                lamk = plsc.load_gather(lam_r, [kk])
                m = jnp.where(sk < thr, jnp.float32(1.0), jnp.float32(0.0))
                wc = lamk * m
                wf = m - wc
                for cidx in range(_D // _L):
                    sl = pl.ds(cidx * _L, _L)
                    obb[k, sl] = wf * cbb[2 * k, sl] + wc * cbb[2 * k + 1, sl]
                return cc

            lax.fori_loop(0, _KT, _row, 0)

        @pl.when(nt > 0)
        def _():
            _issue(0, cb0, ga0)

        def _pair(p, c):
            for b2 in range(2):
                i = p * 2 + b2
                cbb, obb, ga, wb = bufs[b2]
                ncbb, nobb, nga, nwb = bufs[1 - b2]

                @pl.when(i < nt)
                def _():
                    @pl.when(i + 1 < nt)
                    def _():
                        _issue(i + 1, ncbb, nga)

                    _wait_gather(cbb, ga)

                    @pl.when(i >= 2)
                    def _():
                        _wait_wb(obb, wb)

                    _blend(i, cbb, obb)
                    koff = i * stride + half * _KT
                    pltpu.async_copy(obb, out_hbm.at[pl.ds(obase + koff, _KT)], wb)
            return c

        lax.fori_loop(0, (nt + 1) // 2, _pair, 0)

        @pl.when(nt >= 2)
        def _():
            _wait_wb(ob0, wb0)
            _wait_wb(ob1, wb1)

        @pl.when(nt == 1)
        def _():
            _wait_wb(ob0, wb0)

        def _zdrain(i, c):
            pltpu.make_async_copy(zero_v, out_hbm.at[pl.ds(obase, _KT)], zsem).wait()
            return c

        lax.fori_loop(nt, tiles_per_half, _zdrain, 0)
        for j in range(zper_half // _KT):
            pltpu.make_async_copy(zero_v, out_hbm.at[pl.ds(obase, _KT)], zsem).wait()

    out = _k(xf, len_seq, jnp.asarray(sg_np), jnp.asarray(sgi_np), jnp.asarray(lm_np))
    return out.reshape(_B, _MAX_LEN_PAD, _D)


# speedup vs baseline: 2.0849x; 2.0849x over previous
import functools

import numpy as np
import jax
import jax.numpy as jnp
from jax import lax
from jax.experimental import pallas as pl
from jax.experimental.pallas import tpu as pltpu
from jax.experimental.pallas import tpu_sc as plsc

_MIN_LEN_SEG = 19
_MAX_LEN_SEG = 32
_MAX_LEN_SEQ = 2048
_MAX_LEN_PAD = 4096
_MAX_NUM_SEG = _MAX_LEN_SEQ // _MIN_LEN_SEG + 1
_B = 16
_D = 512

_NC, _NS, _L = 2, 16, 16
_KT = 32
_ZB = 128


_ROT = ((13, 15, 26, 6), (17, 29, 16, 24))


def _fry2(k0, k1, x0, x1):
    x0 = x0.astype(np.uint32).copy()
    x1 = x1.astype(np.uint32).copy()
    ks = [np.uint32(k0), np.uint32(k1),
          np.uint32(np.uint32(k0) ^ np.uint32(k1) ^ np.uint32(0x1BD11BDA))]
    with np.errstate(over="ignore"):
        x0 += ks[0]
        x1 += ks[1]
        for i in range(5):
            for r in _ROT[i % 2]:
                x0 += x1
                x1 = (x1 << np.uint32(r)) | (x1 >> np.uint32(32 - r))
                x1 ^= x0
            x0 += ks[(i + 1) % 3]
            x1 += ks[(i + 2) % 3] + np.uint32(i + 1)
    return x0, x1


def _fry_split(k0, k1, num=2):
    y0, y1 = _fry2(k0, k1, np.zeros(num, np.uint32),
                   np.arange(num, dtype=np.uint32))
    return np.stack([y0, y1], axis=1)


def _fry_bits(k0, k1, n):
    y0, y1 = _fry2(k0, k1, np.zeros(n, np.uint32), np.arange(n, dtype=np.uint32))
    return y0 ^ y1


def _fry_uniform(k0, k1, n):
    bits = _fry_bits(k0, k1, n)
    f = ((bits >> np.uint32(9)) | np.uint32(0x3F800000)).view(np.float32)
    return np.maximum(np.float32(0.0), f - np.float32(1.0))


def _fry_randint(k0, k1, n, minval, maxval):
    keys = _fry_split(k0, k1)
    u = _fry_bits(keys[0, 0], keys[0, 1], n)
    v = _fry_bits(keys[1, 0], keys[1, 1], n)
    span = np.uint32(maxval - minval)
    with np.errstate(over="ignore"):
        mult = ((np.uint32(65536) % span) ** 2) % span
        out = ((u % span) * mult + (v % span)) % span
    return minval + out.astype(np.int32)


@functools.lru_cache(maxsize=None)
def _consts():
    n = _B * _MAX_NUM_SEG
    keys = _fry_split(0, 42)
    scales = _fry_uniform(keys[0, 0], keys[0, 1], n) + np.float32(0.5)
    len_seg = _fry_randint(keys[1, 0], keys[1, 1], n,
                           _MIN_LEN_SEG, _MAX_LEN_SEG).reshape(n, 1)
    j = np.arange(_MAX_LEN_SEG * 2, dtype=np.int32)[None, :]
    idx_scaled = j.astype(np.float32) / scales[:, None]
    idx_fl = np.floor(idx_scaled)
    lam = (idx_scaled - idx_fl).astype(np.float32)
    static_mask = idx_fl < (len_seg.astype(np.float32) - np.float32(1.0))
    off = np.cumsum(len_seg.reshape(_B, -1), axis=-1)
    off = np.pad(off[:, :-1], ((0, 0), (1, 0))).reshape(-1, 1)
    src = (idx_fl + off.astype(np.float32)).astype(np.int32)
    grid = _MAX_NUM_SEG * _MAX_LEN_SEG * 2
    src = src.reshape(_B, grid)
    lam = lam.reshape(_B, grid)
    static_mask = static_mask.reshape(_B, grid)
    rows, lams, counts = [], [], []
    for b in range(_B):
        sel = static_mask[b] & (src[b] < _MAX_LEN_SEQ - 2)
        rows.append(src[b][sel])
        lams.append(lam[b][sel])
        counts.append(int(sel.sum()))
    K = -(-max(counts) // 128) * 128
    sg = np.full((_B, K), 4000, np.int32)
    lm = np.zeros((_B, K), np.float32)
    for b in range(_B):
        sg[b, : counts[b]] = rows[b]
        lm[b, : counts[b]] = lams[b]
    sg = sg + (np.arange(_B, dtype=np.int32) * _MAX_LEN_PAD)[:, None]
    return sg, sg + 1, lm, K


def kernel(x, len_seq):
    sg_np, sc_np, lm_np, K = _consts()
    tiles_per_half = (K // _KT) // _NC
    zrows = _MAX_LEN_PAD - K
    zper_half = zrows // _NC

    xf = x.reshape(_B * _MAX_LEN_PAD, _D)
    mesh = plsc.VectorSubcoreMesh(core_axis_name="c", subcore_axis_name="s")

    @functools.partial(
        pl.kernel,
        mesh=mesh,
        compiler_params=pltpu.CompilerParams(needs_layout_passes=False),
        out_type=jax.ShapeDtypeStruct((_B * _MAX_LEN_PAD, _D), jnp.float32),
        scratch_types=[
            pltpu.VMEM((K,), jnp.int32),
            pltpu.VMEM((K,), jnp.int32),
            pltpu.VMEM((K,), jnp.float32),
            pltpu.VMEM((_KT, _D), jnp.float32),
            pltpu.VMEM((_KT, _D), jnp.float32),
            pltpu.VMEM((_KT, _D), jnp.float32),
            pltpu.VMEM((_KT, _D), jnp.float32),
            pltpu.VMEM((_KT, _D), jnp.float32),
            pltpu.VMEM((_L,), jnp.int32),
            pltpu.SemaphoreType.DMA,
            pltpu.SemaphoreType.DMA,
            pltpu.SemaphoreType.DMA,
            pltpu.SemaphoreType.DMA,
            pltpu.SemaphoreType.DMA,
            pltpu.SemaphoreType.DMA,
            pltpu.SemaphoreType.DMA,
        ],
    )
    def _k(xf_hbm, lseq_hbm, sfl_hbm, scl_hbm, lam_hbm, out_hbm,
           sfl_r, scl_r, lam_r, fl0, cl0, fl1, cl1, zero_v, lseq_v,
           ga0, gb0, ga1, gb1, wb0, wb1, csem):
        b = lax.axis_index("s")
        half = lax.axis_index("c")
        zsem = csem
        pltpu.async_copy(lseq_hbm, lseq_v, csem)
        pltpu.async_copy(sfl_hbm.at[b], sfl_r, csem)
        pltpu.async_copy(scl_hbm.at[b], scl_r, csem)
        pltpu.async_copy(lam_hbm.at[b], lam_r, csem)

        zeros16 = jnp.zeros((_L,), jnp.float32)

        def _memset_row(r, c):
            for cidx in range(_D // _L):
                zero_v[r, pl.ds(cidx * _L, _L)] = zeros16
            return c

        lax.fori_loop(0, _KT, _memset_row, 0)

        pltpu.make_async_copy(lseq_hbm, lseq_v, csem).wait()
        pltpu.make_async_copy(sfl_hbm.at[b], sfl_r, csem).wait()
        pltpu.make_async_copy(scl_hbm.at[b], scl_r, csem).wait()
        pltpu.make_async_copy(lam_hbm.at[b], lam_r, csem).wait()

        thr = (plsc.load_gather(lseq_v, [jnp.full((_L,), b, jnp.int32)])
               + b * _MAX_LEN_PAD - 1)

        def _cnt(i, acc):
            s16 = sfl_r[pl.ds(i * _L, _L)]
            return acc + jnp.where(s16 < thr, jnp.int32(1), jnp.int32(0))

        nvec = lax.fori_loop(0, K // _L, _cnt, jnp.zeros((_L,), jnp.int32))
        n_b = jnp.sum(nvec)
        stride = _NC * _KT
        nt = jnp.minimum((jnp.maximum(n_b - half * _KT, 0) + (stride - 1))
                         // stride, tiles_per_half)

        obase = b * _MAX_LEN_PAD

        def _ztile(i, c):
            koff = i * stride + half * _KT
            pltpu.async_copy(zero_v, out_hbm.at[pl.ds(obase + koff, _KT)], zsem)
            return c

        lax.fori_loop(nt, tiles_per_half, _ztile, 0)

        zbase = obase + K + half * zper_half
        for j in range(zper_half // _KT):
            pltpu.async_copy(zero_v, out_hbm.at[pl.ds(zbase + j * _KT, _KT)], zsem)

        bufs = ((fl0, cl0, ga0, gb0, wb0), (fl1, cl1, ga1, gb1, wb1))

        def _issue(i, flb, clb, ga, gb):
            koff = i * stride + half * _KT
            pltpu.async_copy(xf_hbm.at[sfl_r.at[pl.ds(koff, _KT)]], flb, ga)
            pltpu.async_copy(xf_hbm.at[scl_r.at[pl.ds(koff, _KT)]], clb, gb)

        def _wait_gather(flb, clb, ga, gb):
            pltpu.make_async_copy(xf_hbm.at[sfl_r.at[pl.ds(0, _KT)]], flb, ga).wait()
            pltpu.make_async_copy(xf_hbm.at[scl_r.at[pl.ds(0, _KT)]], clb, gb).wait()

        def _wait_wb(flb, wb):
            pltpu.make_async_copy(flb, out_hbm.at[pl.ds(0, _KT)], wb).wait()

        def _blend(i, flb, clb):
            koff = i * stride + half * _KT

            def _row(k, cc):
                kk = jnp.full((_L,), koff + k, jnp.int32)
                sk = plsc.load_gather(sfl_r, [kk])
                lamk = plsc.load_gather(lam_r, [kk])
                m = jnp.where(sk < thr, jnp.float32(1.0), jnp.float32(0.0))
                wc = lamk * m
                wf = m - wc
                for cidx in range(_D // _L):
                    sl = pl.ds(cidx * _L, _L)
                    flb[k, sl] = wf * flb[k, sl] + wc * clb[k, sl]
                return cc

            lax.fori_loop(0, _KT, _row, 0)

        @pl.when(nt > 0)
        def _():
            _issue(0, fl0, cl0, ga0, gb0)

        def _pair(p, c):
            for b2 in range(2):
                i = p * 2 + b2
                flb, clb, ga, gb, wb = bufs[b2]
                nflb, nclb, nga, ngb, nwb = bufs[1 - b2]

                @pl.when(i < nt)
                def _():
                    @pl.when(i + 1 < nt)
                    def _():
                        @pl.when(i >= 1)
                        def _():
                            _wait_wb(nflb, nwb)

                        _issue(i + 1, nflb, nclb, nga, ngb)

                    _wait_gather(flb, clb, ga, gb)
                    _blend(i, flb, clb)
                    koff = i * stride + half * _KT
                    pltpu.async_copy(flb, out_hbm.at[pl.ds(obase + koff, _KT)], wb)
            return c

        lax.fori_loop(0, (nt + 1) // 2, _pair, 0)

        @pl.when(nt >= 2)
        def _():
            _wait_wb(fl0, wb0)
            _wait_wb(fl1, wb1)

        @pl.when(nt == 1)
        def _():
            _wait_wb(fl0, wb0)

        def _zdrain(i, c):
            pltpu.make_async_copy(zero_v, out_hbm.at[pl.ds(obase, _KT)], zsem).wait()
            return c

        lax.fori_loop(nt, tiles_per_half, _zdrain, 0)
        for j in range(zper_half // _KT):
            pltpu.make_async_copy(zero_v, out_hbm.at[pl.ds(obase, _KT)], zsem).wait()

    out = _k(xf, len_seq, jnp.asarray(sg_np), jnp.asarray(sc_np), jnp.asarray(lm_np))
    return out.reshape(_B, _MAX_LEN_PAD, _D)
